# baseline (speedup 1.0000x reference)
import functools

import jax
import jax.numpy as jnp
from jax import lax
from jax.experimental import pallas as pl
from jax.experimental.pallas import tpu as pltpu

D = 128


def _mm_kernel(x_ref, w_ref, o_ref):
    o_ref[...] = jnp.dot(x_ref[...], w_ref[...], preferred_element_type=jnp.float32)


def _matmul(x, w, block=1024):
    n, d = x.shape
    k = w.shape[1]
    grid = (n + block - 1) // block
    return pl.pallas_call(
        _mm_kernel,
        grid=(grid,),
        in_specs=[
            pl.BlockSpec((block, d), lambda i: (i, 0)),
            pl.BlockSpec((d, k), lambda i: (0, 0)),
        ],
        out_specs=pl.BlockSpec((block, k), lambda i: (i, 0)),
        out_shape=jax.ShapeDtypeStruct((n, k), jnp.float32),
    )(x, w)


def _scatter(src, idx, size):
    return jnp.zeros((size, src.shape[1]), src.dtype).at[idx].add(src)


def kernel(edge_attr0, edge_attr1, edge_attr2, edge_index0, edge_index, edge_index2,
           triangle_0_1_1, triangle_1_1_1, triangle_1_1_2, triangle_1_2_2, triangle_2_2_2,
           inverse_edge_1, inverse_edge_2,
           proj0_W, proj0_b, proj1_W, proj1_b, proj2_W, proj2_b,
           mlp0_W1, mlp0_b1, mlp0_g, mlp0_beta, mlp0_W2, mlp0_b2,
           mlp1_W1, mlp1_b1, mlp1_g, mlp1_beta, mlp1_W2, mlp1_b2,
           mlp2_W1, mlp2_b1, mlp2_g, mlp2_beta, mlp2_W2, mlp2_b2,
           norm0_g, norm0_beta, norm1_g, norm1_beta, norm2_g, norm2_beta,
           eps0, eps1, eps2):
    e0, e1, e2 = edge_attr0, edge_attr1, edge_attr2
    num0, num1, num2 = e0.shape[0], e1.shape[0], e2.shape[0]

    p1cat = _matmul(e1, jnp.concatenate([2.0 * proj0_W, proj1_W, proj2_W], axis=1))
    A0, B1, B2 = p1cat[:, :D], p1cat[:, D:2 * D], p1cat[:, 2 * D:]
    p2cat = _matmul(e2, jnp.concatenate([proj1_W, proj2_W], axis=1))
    C1, C2 = p2cat[:, :D], p2cat[:, D:]

    ij011, ik011 = triangle_0_1_1[0], triangle_0_1_1[1]
    ij111, ik111, kj111 = triangle_1_1_1[0], triangle_1_1_1[1], triangle_1_1_1[2]
    ij112, ik112, kj112 = triangle_1_1_2[0], triangle_1_1_2[1], triangle_1_1_2[2]
    ij122, ik122, kj122 = triangle_1_2_2[0], triangle_1_2_2[1], triangle_1_2_2[2]
    ij222, ik222, kj222 = triangle_2_2_2[0], triangle_2_2_2[1], triangle_2_2_2[2]

    x011 = jax.nn.relu(A0[ik011] + proj0_b)
    acc0 = _scatter(x011, ij011, num0)

    x111 = jax.nn.relu(B1[ik111] + B1[kj111] + proj1_b)
    acc1 = _scatter(x111, ij111, num1)
    x112 = jax.nn.relu(B1[ik112] + C1[kj112] + proj1_b)
    a112 = _scatter(x112, ij112, num1)
    acc1 = acc1 + a112 + a112[inverse_edge_1]
    x122 = jax.nn.relu(C1[ik122] + C1[kj122] + proj1_b)
    acc1 = acc1 + _scatter(x122, ij122, num1)

    x211 = jax.nn.relu(B2[ij112] + B2[ik112] + proj2_b)
    acc2 = _scatter(x211, kj112, num2)
    x212 = jax.nn.relu(B2[ij122] + C2[kj122] + proj2_b)
    a212 = _scatter(x212, ik122, num2)
    acc2 = acc2 + a212 + a212[inverse_edge_2]
    x222 = jax.nn.relu(C2[ik222] + C2[kj222] + proj2_b)
    acc2 = acc2 + _scatter(x222, ij222, num2)

    def _bn(x, g, b):
        m = jnp.mean(x, axis=0, keepdims=True)
        v = jnp.var(x, axis=0, keepdims=True)
        return (x - m) / jnp.sqrt(v + 1e-5) * g + b

    def _head(x, W1, b1, g, bt, W2, b2, ng, nbt):
        h = _matmul(x, W1) + b1
        h = jax.nn.relu(_bn(h, g, bt))
        o = _matmul(h, W2) + b2
        return _bn(o, ng, nbt)

    out0 = _head((1.0 + eps0) * e0 + acc0, mlp0_W1, mlp0_b1, mlp0_g, mlp0_beta,
                 mlp0_W2, mlp0_b2, norm0_g, norm0_beta)
    out1 = _head((1.0 + eps1) * e1 + acc1, mlp1_W1, mlp1_b1, mlp1_g, mlp1_beta,
                 mlp1_W2, mlp1_b2, norm1_g, norm1_beta)
    out2 = _head((1.0 + eps2) * e2 + acc2, mlp2_W1, mlp2_b1, mlp2_g, mlp2_beta,
                 mlp2_W2, mlp2_b2, norm2_g, norm2_beta)
    return out0, out1, out2
